# Initial kernel scaffold; baseline (speedup 1.0000x reference)
import jax
import jax.numpy as jnp
from jax.experimental import pallas as pl


def kernel(user, pos, neg, random_neg, x, edge_index, batch, item_table, ggc_weight, gru_w_ih, gru_w_hh, gru_b_ih, gru_b_hh, w1_w, w1_b, w2_w, w2_b, q_w, q_b, w3_w, w3_b):
    raise NotImplementedError("write your pallas kernel here")


# speedup vs baseline: 4.0894x; 4.0894x over previous
import functools

import jax
import jax.numpy as jnp
from jax import lax
from jax.experimental import pallas as pl
from jax.experimental.pallas import tpu as pltpu
from jax.experimental.pallas import tpu_sc as plsc

N = 10000
E = 320000
B = 512
D = 128
NUM_ITEM = 100000
WEIGHT_DECAY_C = 1e-05

NW = 32
NPAD = 10240
RPT = NPAD // 16
K = 128
EPW = 10112
KC = EPW // K
EPAD = EPW * NW
G = 12288
GC = G // (NW * K)
RB = 1280
NB = NPAD // RB

@functools.lru_cache(maxsize=None)
def _mesh():
  return plsc.VectorSubcoreMesh(
      core_axis_name="c", subcore_axis_name="s", num_cores=2, num_subcores=16)


@functools.lru_cache(maxsize=None)
def _gather_kernel(total, chunks, k):
  @functools.partial(
      pl.kernel,
      out_type=jax.ShapeDtypeStruct((total, D), jnp.float32),
      mesh=_mesh(),
      scratch_types=[
          pltpu.VMEM((chunks, k), jnp.int32),
          pltpu.VMEM((k, D), jnp.float32),
          pltpu.SemaphoreType.DMA,
      ],
  )
  def gk(table, idx, out, idxv, rows, sem):
    wid = lax.axis_index("s") * 2 + lax.axis_index("c")
    pltpu.sync_copy(idx.at[wid], idxv)
    for j in range(chunks):
      pltpu.async_copy(table.at[idxv.at[j]], rows, sem).wait()
      pltpu.sync_copy(rows, out.at[pl.ds((wid * chunks + j) * k, k)])

  return gk


@functools.lru_cache(maxsize=None)
def _edge_kernel():

  @functools.partial(
      pl.kernel,
      out_type=jax.ShapeDtypeStruct((2, NPAD, D), jnp.float32),
      mesh=_mesh(),
      scratch_types=[
          pltpu.VMEM((KC, K), jnp.int32),
          pltpu.VMEM((KC, K), jnp.int32),
          pltpu.VMEM((K, D), jnp.float32),
          pltpu.VMEM_SHARED((NPAD, D), jnp.float32),
          pltpu.SemaphoreType.DMA,
      ],
  )
  def ek(m, src2, dst2, zrows, out, srcv, dstv, rows, acc, sem):
    cid = lax.axis_index("c")
    sid = lax.axis_index("s")
    wid = sid * 2 + cid
    pltpu.sync_copy(zrows, acc.at[pl.ds(sid * RPT, RPT)])
    pltpu.sync_copy(src2.at[wid], srcv)
    pltpu.sync_copy(dst2.at[wid], dstv)
    plsc.subcore_barrier()

    @pl.loop(0, KC)
    def _(j):
      pltpu.async_copy(m.at[srcv.at[j]], rows, sem).wait()
      pltpu.sync_copy(rows, acc.at[dstv.at[j]], add=True)

    plsc.subcore_barrier()
    pltpu.sync_copy(acc.at[pl.ds(sid * RPT, RPT)],
                    out.at[cid, pl.ds(sid * RPT, RPT)])

  return ek


def _gather_big(table, idx2d):
  return _gather_kernel(G, GC, K)(table, idx2d)


def _gather_vn(table, idx2d):
  return _gather_kernel(B, 1, 16)(table, idx2d)


def _edge_agg(m, src2, dst2, zrows):
  return _edge_kernel()(m, src2, dst2, zrows)


def _mm_body(h_ref, w_ref, o_ref):
  o_ref[...] = jnp.dot(h_ref[...], w_ref[...],
                       preferred_element_type=jnp.float32)


def _matmul(h, w):
  return pl.pallas_call(
      _mm_body,
      grid=(NB,),
      in_specs=[
          pl.BlockSpec((RB, D), lambda i: (i, 0)),
          pl.BlockSpec((D, D), lambda i: (0, 0)),
      ],
      out_specs=pl.BlockSpec((RB, D), lambda i: (i, 0)),
      out_shape=jax.ShapeDtypeStruct((NPAD, D), jnp.float32),
  )(h, w)


def _gru_body_fused(agg_ref, h_ref, wih_ref, whh_ref, bih_ref, bhh_ref,
                    wn_ref, o_ref, mn_ref):
  hn = _gru_core(agg_ref, h_ref, wih_ref, whh_ref, bih_ref, bhh_ref)
  o_ref[...] = hn
  mn_ref[...] = jnp.dot(hn, wn_ref[...], preferred_element_type=jnp.float32)


def _gru_body_plain(agg_ref, h_ref, wih_ref, whh_ref, bih_ref, bhh_ref, o_ref):
  o_ref[...] = _gru_core(agg_ref, h_ref, wih_ref, whh_ref, bih_ref, bhh_ref)


def _gru_core(agg_ref, h_ref, wih_ref, whh_ref, bih_ref, bhh_ref):
  m = agg_ref[0] + agg_ref[1]
  h = h_ref[...]
  gi = jnp.dot(m, wih_ref[...], preferred_element_type=jnp.float32) + bih_ref[...]
  gh = jnp.dot(h, whh_ref[...], preferred_element_type=jnp.float32) + bhh_ref[...]
  r = jax.nn.sigmoid(gi[:, :D] + gh[:, :D])
  z = jax.nn.sigmoid(gi[:, D:2 * D] + gh[:, D:2 * D])
  n = jnp.tanh(gi[:, 2 * D:] + r * gh[:, 2 * D:])
  return (1.0 - z) * n + z * h


def _gru(agg2, h, wihT, whhT, bih2, bhh2, w_next):
  full = lambda i: (0, 0)
  in_specs = [
      pl.BlockSpec((2, RB, D), lambda i: (0, i, 0)),
      pl.BlockSpec((RB, D), lambda i: (i, 0)),
      pl.BlockSpec((D, 3 * D), full),
      pl.BlockSpec((D, 3 * D), full),
      pl.BlockSpec((1, 3 * D), full),
      pl.BlockSpec((1, 3 * D), full),
  ]
  args = [agg2, h, wihT, whhT, bih2, bhh2]
  hs = jax.ShapeDtypeStruct((NPAD, D), jnp.float32)
  if w_next is None:
    return pl.pallas_call(
        _gru_body_plain,
        grid=(NB,),
        in_specs=in_specs,
        out_specs=pl.BlockSpec((RB, D), lambda i: (i, 0)),
        out_shape=hs,
    )(*args)
  return pl.pallas_call(
      _gru_body_fused,
      grid=(NB,),
      in_specs=in_specs + [pl.BlockSpec((D, D), full)],
      out_specs=[pl.BlockSpec((RB, D), lambda i: (i, 0))] * 2,
      out_shape=[hs, hs],
  )(*args, w_next)


def _ends_body(b_ref, o_ref):
  acc = jnp.zeros((B, 1), jnp.float32)
  for r in range(8):
    row = b_ref[r, :].reshape(1, RB)
    io = lax.broadcasted_iota(jnp.int32, (B, RB), 0)
    acc += jnp.sum((row <= io).astype(jnp.float32), axis=1, keepdims=True)
  ends = jnp.maximum(acc - 1.0, 0.0).astype(jnp.int32)
  o_ref[...] = jnp.broadcast_to(ends, (B, 128))


def _ends(batch2d):
  return pl.pallas_call(
      _ends_body,
      grid=(1,),
      in_specs=[pl.BlockSpec((8, RB), lambda i: (0, 0))],
      out_specs=pl.BlockSpec((B, 128), lambda i: (0, 0)),
      out_shape=jax.ShapeDtypeStruct((B, 128), jnp.int32),
  )(batch2d)


def _tail_body(h_ref, b_ref, vn_ref, pe_ref, ne_ref, w1t_ref, w2t_ref, bb_ref,
               qr_ref, qb_ref, w3a_ref, w3b_ref, b3_ref, o_ref, t1, s_g):
  i = pl.program_id(0)

  @pl.when(i == 0)
  def _():
    vn = jax.nn.relu(vn_ref[...])
    t1[...] = jnp.dot(vn, w1t_ref[...], preferred_element_type=jnp.float32)
    s_g[...] = jnp.zeros((B, D), jnp.float32)

  h2 = jax.nn.relu(h_ref[...])
  bt = b_ref[...].reshape(1, RB)
  io = lax.broadcasted_iota(jnp.int32, (B, RB), 0)
  pt = (bt == io).astype(jnp.float32)
  v_rep = lax.dot_general(pt, t1[...], (((0,), (0,)), ((), ())),
                          preferred_element_type=jnp.float32)
  pre = jax.nn.sigmoid(
      v_rep + jnp.dot(h2, w2t_ref[...], preferred_element_type=jnp.float32)
      + bb_ref[...])
  alpha = jnp.sum(pre * qr_ref[...], axis=1, keepdims=True) + qb_ref[...]
  s_g[...] += jnp.dot(pt, alpha * h2, preferred_element_type=jnp.float32)

  @pl.when(i == NB - 1)
  def _():
    vn = jax.nn.relu(vn_ref[...])
    s_h = (jnp.dot(vn, w3a_ref[...], preferred_element_type=jnp.float32)
           + jnp.dot(s_g[...], w3b_ref[...], preferred_element_type=jnp.float32)
           + b3_ref[...])
    pe = pe_ref[...]
    ne = ne_ref[...]
    pp = jnp.sum(s_h * pe, axis=1)
    pn = jnp.sum(s_h * ne, axis=1)
    dlt = pp - pn
    ls = -(jnp.maximum(-dlt, 0.0) + jnp.log1p(jnp.exp(-jnp.abs(dlt))))
    orig = -jnp.sum(ls)
    raw = jnp.sum(s_h * s_h) + jnp.sum(pe * pe) + jnp.sum(ne * ne)
    nr = WEIGHT_DECAY_C * raw
    lane = lax.broadcasted_iota(jnp.int32, (8, 128), 1)
    vals = jnp.where(lane == 0, orig + nr,
                     jnp.where(lane == 1, orig,
                               jnp.where(lane == 2, nr, 0.0)))
    o_ref[...] = vals


def _tail(h_final, batch3d, vn_raw, pos_e, neg_e, w1t, w2t, bb, qrow, qb,
          w3at, w3bt, b3):
  full = lambda i: (0, 0)
  return pl.pallas_call(
      _tail_body,
      grid=(NB,),
      in_specs=[
          pl.BlockSpec((RB, D), lambda i: (i, 0)),
          pl.BlockSpec((1, 1, RB), lambda i: (i, 0, 0)),
          pl.BlockSpec((B, D), full),
          pl.BlockSpec((B, D), full),
          pl.BlockSpec((B, D), full),
          pl.BlockSpec((D, D), full),
          pl.BlockSpec((D, D), full),
          pl.BlockSpec((1, D), full),
          pl.BlockSpec((1, D), full),
          pl.BlockSpec((1, 1), full),
          pl.BlockSpec((D, D), full),
          pl.BlockSpec((D, D), full),
          pl.BlockSpec((1, D), full),
      ],
      out_specs=pl.BlockSpec((8, 128), full),
      out_shape=jax.ShapeDtypeStruct((8, 128), jnp.float32),
      scratch_shapes=[
          pltpu.VMEM((B, D), jnp.float32),
          pltpu.VMEM((B, D), jnp.float32),
      ],
  )(h_final, batch3d, vn_raw, pos_e, neg_e, w1t, w2t, bb, qrow, qb,
    w3at, w3bt, b3)


def kernel(user, pos, neg, random_neg, x, edge_index, batch, item_table,
           ggc_weight, gru_w_ih, gru_w_hh, gru_b_ih, gru_b_hh,
           w1_w, w1_b, w2_w, w2_b, q_w, q_b, w3_w, w3_b):
  f32 = jnp.float32
  i32 = jnp.int32
  x_idx = jnp.clip(x.astype(i32) - 1, 0, NUM_ITEM - 1).reshape(N)
  gidx = jnp.concatenate([
      x_idx,
      jnp.zeros((NPAD - N,), i32),
      pos.astype(i32),
      random_neg.astype(i32),
      jnp.zeros((G - NPAD - 2 * B,), i32),
  ])
  table = item_table.astype(f32)
  gout = _gather_big(table, gidx.reshape(NW, GC, K))
  h0 = gout[:NPAD]
  pos_e = gout[NPAD:NPAD + B]
  neg_e = gout[NPAD + B:NPAD + 2 * B]

  src = edge_index[0].astype(i32)
  dst = edge_index[1].astype(i32)
  epad = jnp.full((EPAD - E,), N, i32)
  src2 = jnp.concatenate([src, epad]).reshape(NW, KC, K)
  dst2 = jnp.concatenate([dst, epad]).reshape(NW, KC, K)
  zrows = jnp.zeros((RPT, D), f32)

  ggc = ggc_weight.astype(f32)
  wihT = gru_w_ih.astype(f32).T
  whhT = gru_w_hh.astype(f32).T
  bih2 = gru_b_ih.astype(f32).reshape(1, 3 * D)
  bhh2 = gru_b_hh.astype(f32).reshape(1, 3 * D)

  m0 = _matmul(h0, ggc[0])
  agg0 = _edge_agg(m0, src2, dst2, zrows)
  h1, m1 = _gru(agg0, h0, wihT, whhT, bih2, bhh2, ggc[1])
  agg1 = _edge_agg(m1, src2, dst2, zrows)
  h_fin = _gru(agg1, h1, wihT, whhT, bih2, bhh2, None)

  batch_p = jnp.concatenate([batch.astype(i32), jnp.full((NPAD - N,), B, i32)])
  ends128 = _ends(batch_p.reshape(8, RB))
  vn_raw = _gather_vn(h_fin, ends128[:, 0].reshape(NW, 1, 16))

  loss = _tail(
      h_fin, batch_p.reshape(8, 1, RB), vn_raw, pos_e, neg_e,
      w1_w.astype(f32).T, w2_w.astype(f32).T,
      (w1_b + w2_b).astype(f32).reshape(1, D),
      q_w.astype(f32).reshape(1, D),
      q_b.astype(f32).reshape(1, 1),
      w3_w[:, :D].astype(f32).T, w3_w[:, D:].astype(f32).T,
      w3_b.astype(f32).reshape(1, D),
  )
  nr = loss[0, 2]
  return (loss[0, 0], loss[0, 1], nr, nr, nr)
